# Initial kernel scaffold; baseline (speedup 1.0000x reference)
import functools

import jax
import jax.numpy as jnp
from jax import lax
from jax.experimental import pallas as pl
from jax.experimental.pallas import tpu as pltpu
from jax.experimental.pallas import tpu_sc as plsc

_NUM_HEADS = 8
_POS_PAD = 16


def _proj_body(x_ref, pos_ref, wq_ref, bq_ref, wkv_ref, bkv_ref, q_ref, tab_ref):
    xb = x_ref[...]
    q_ref[...] = jnp.dot(xb, wq_ref[...], preferred_element_type=jnp.float32) + bq_ref[...]
    kv = jnp.dot(xb, wkv_ref[...], preferred_element_type=jnp.float32) + bkv_ref[...]
    tab_ref[...] = jnp.concatenate([kv, pos_ref[...]], axis=1)


def _erf(x):
    a1, a2, a3, a4, a5 = (0.254829592, -0.284496736, 1.421413741,
                          -1.453152027, 1.061405429)
    p = 0.3275911
    s = jnp.sign(x)
    ax = jnp.abs(x)
    t = 1.0 / (1.0 + p * ax)
    poly = t * (a1 + t * (a2 + t * (a3 + t * (a4 + t * a5))))
    return s * (1.0 - poly * jnp.exp(-ax * ax))


def _gelu(x):
    return 0.5 * x * (1.0 + _erf(x * 0.7071067811865476))


def _attn_body(scale, bn, k, c,
               q_ref, x_ref, pos_ref, g_ref, wo_ref, bo_ref, w1_ref, b1_ref,
               w2_ref, b2_ref, ssum_ref, sbce_ref, gm_ref, bt_ref, o_ref):
    e = bn * k
    g = g_ref[...]
    kf = g[:, :c]
    vf = g[:, c:2 * c]
    npos = g[:, 2 * c:]
    q = q_ref[...]
    qrep = jnp.reshape(jnp.broadcast_to(q[:, None, :], (bn, k, c)), (e, c))
    logits = jnp.dot(kf * qrep, ssum_ref[...],
                     preferred_element_type=jnp.float32) * scale
    posrep = jnp.reshape(
        jnp.broadcast_to(pos_ref[...][:, None, :], (bn, k, _POS_PAD)),
        (e, _POS_PAD))
    rel = npos - posrep
    h = _gelu(jnp.dot(rel, w1_ref[...], preferred_element_type=jnp.float32)
              + b1_ref[...])
    z = logits + jnp.dot(h, w2_ref[...], preferred_element_type=jnp.float32) \
        + b2_ref[...]
    z3 = z.reshape(bn, k, _NUM_HEADS)
    zmax = jnp.max(z3, axis=1, keepdims=True)
    ez = jnp.exp(z3 - zmax)
    attn = ez / jnp.sum(ez, axis=1, keepdims=True)
    attn2 = attn.reshape(e, _NUM_HEADS)
    w = jnp.dot(attn2, sbce_ref[...], preferred_element_type=jnp.float32) * vf
    out = jnp.sum(w.reshape(bn, k, c), axis=1)
    y = jnp.dot(out, wo_ref[...], preferred_element_type=jnp.float32) \
        + bo_ref[...] + x_ref[...]
    mean = jnp.mean(y, axis=1, keepdims=True)
    var = jnp.mean((y - mean) ** 2, axis=1, keepdims=True)
    o_ref[...] = (y - mean) * lax.rsqrt(var + 1e-5) * gm_ref[...] + bt_ref[...]


def _sc_gather(tab, idx_flat, n_workers=32, chunk=40):
    e_total = idx_flat.shape[0]
    d = tab.shape[1]
    epw = e_total // n_workers
    assert epw * n_workers == e_total and epw % chunk == 0
    n_chunks = epw // chunk
    mesh = plsc.VectorSubcoreMesh(core_axis_name="c", subcore_axis_name="s")

    @functools.partial(
        pl.kernel,
        out_type=jax.ShapeDtypeStruct((e_total, d), jnp.float32),
        mesh=mesh,
        scratch_types=[
            pltpu.VMEM((epw,), jnp.int32),
            pltpu.VMEM((chunk, d), jnp.float32),
            pltpu.SemaphoreType.DMA,
        ],
    )
    def gather_kernel(tab_hbm, idx_hbm, out_hbm, idx_v, rows_v, sem):
        wid = lax.axis_index("s") * 2 + lax.axis_index("c")
        base = wid * epw
        pltpu.sync_copy(idx_hbm.at[pl.ds(base, epw)], idx_v)

        def body(i, carry):
            off = i * chunk
            pltpu.async_copy(
                tab_hbm.at[idx_v.at[pl.ds(off, chunk)]], rows_v, sem).wait()
            pltpu.sync_copy(rows_v, out_hbm.at[pl.ds(base + off, chunk)])
            return carry

        lax.fori_loop(0, n_chunks, body, 0)

    return gather_kernel(tab, idx_flat)


def kernel(x, pos, idx, Wq, bq, Wk, bk, Wv, bv, Wo, bo, W1, b1, W2, b2,
           gamma, beta):
    n, c = x.shape
    k = idx.shape[1]
    h = _NUM_HEADS
    hd = c // h
    scale = float(hd) ** -0.5
    d_tab = 2 * c + _POS_PAD

    pos_pad = jnp.pad(pos, ((0, 0), (0, _POS_PAD - pos.shape[1])))
    wkv = jnp.concatenate([Wk, Wv], axis=1)
    bkv = jnp.concatenate([bk, bv])[None, :]

    bn_a = 256
    grid_a = (n + bn_a - 1) // bn_a
    q, tab = pl.pallas_call(
        _proj_body,
        grid=(grid_a,),
        in_specs=[
            pl.BlockSpec((bn_a, c), lambda i: (i, 0)),
            pl.BlockSpec((bn_a, _POS_PAD), lambda i: (i, 0)),
            pl.BlockSpec((c, c), lambda i: (0, 0)),
            pl.BlockSpec((1, c), lambda i: (0, 0)),
            pl.BlockSpec((c, 2 * c), lambda i: (0, 0)),
            pl.BlockSpec((1, 2 * c), lambda i: (0, 0)),
        ],
        out_specs=[
            pl.BlockSpec((bn_a, c), lambda i: (i, 0)),
            pl.BlockSpec((bn_a, d_tab), lambda i: (i, 0)),
        ],
        out_shape=[
            jax.ShapeDtypeStruct((n, c), jnp.float32),
            jax.ShapeDtypeStruct((n, d_tab), jnp.float32),
        ],
    )(x, pos_pad, Wq, bq[None, :], wkv, bkv)

    g = _sc_gather(tab, idx.reshape(-1))

    head_ids = jnp.arange(c, dtype=jnp.int32) // hd
    ssum = (head_ids[:, None] == jnp.arange(h, dtype=jnp.int32)[None, :]
            ).astype(jnp.float32)
    sbce = ssum.T
    w1p = jnp.pad(W1, ((0, _POS_PAD - W1.shape[0]), (0, 0)))

    bn_c = 128
    grid_c = (n + bn_c - 1) // bn_c
    y = pl.pallas_call(
        functools.partial(_attn_body, scale, bn_c, k, c),
        grid=(grid_c,),
        in_specs=[
            pl.BlockSpec((bn_c, c), lambda i: (i, 0)),
            pl.BlockSpec((bn_c, c), lambda i: (i, 0)),
            pl.BlockSpec((bn_c, _POS_PAD), lambda i: (i, 0)),
            pl.BlockSpec((bn_c * k, d_tab), lambda i: (i, 0)),
            pl.BlockSpec((c, c), lambda i: (0, 0)),
            pl.BlockSpec((1, c), lambda i: (0, 0)),
            pl.BlockSpec((_POS_PAD, 64), lambda i: (0, 0)),
            pl.BlockSpec((1, 64), lambda i: (0, 0)),
            pl.BlockSpec((64, h), lambda i: (0, 0)),
            pl.BlockSpec((1, h), lambda i: (0, 0)),
            pl.BlockSpec((c, h), lambda i: (0, 0)),
            pl.BlockSpec((h, c), lambda i: (0, 0)),
            pl.BlockSpec((1, c), lambda i: (0, 0)),
            pl.BlockSpec((1, c), lambda i: (0, 0)),
        ],
        out_specs=pl.BlockSpec((bn_c, c), lambda i: (i, 0)),
        out_shape=jax.ShapeDtypeStruct((n, c), jnp.float32),
    )(q, x, pos_pad, g, Wo, bo[None, :], w1p, b1[None, :], W2, b2[None, :],
      ssum, sbce, gamma[None, :], beta[None, :])
    return y


# speedup vs baseline: 1.6786x; 1.6786x over previous
import functools

import jax
import jax.numpy as jnp
from jax import lax
from jax.experimental import pallas as pl
from jax.experimental.pallas import tpu as pltpu
from jax.experimental.pallas import tpu_sc as plsc

_NUM_HEADS = 8
_POS_PAD = 16


def _proj_body(x_ref, pos_ref, wq_ref, bq_ref, wkv_ref, bkv_ref, q_ref, tab_ref):
    xb = x_ref[...]
    q_ref[...] = jnp.dot(xb, wq_ref[...], preferred_element_type=jnp.float32) + bq_ref[...]
    kv = jnp.dot(xb, wkv_ref[...], preferred_element_type=jnp.float32) + bkv_ref[...]
    tab_ref[...] = jnp.concatenate([kv, pos_ref[...]], axis=1)


def _erf(x):
    a1, a2, a3, a4, a5 = (0.254829592, -0.284496736, 1.421413741,
                          -1.453152027, 1.061405429)
    p = 0.3275911
    s = jnp.sign(x)
    ax = jnp.abs(x)
    t = 1.0 / (1.0 + p * ax)
    poly = t * (a1 + t * (a2 + t * (a3 + t * (a4 + t * a5))))
    return s * (1.0 - poly * jnp.exp(-ax * ax))


def _gelu(x):
    return 0.5 * x * (1.0 + _erf(x * 0.7071067811865476))


def _attn_body(scale, bn, k, c,
               q_ref, x_ref, pos_ref, g_ref, wo_ref, bo_ref, w1_ref, b1_ref,
               w2_ref, b2_ref, ssum_ref, sbce_ref, gm_ref, bt_ref, o_ref):
    e = bn * k
    g = g_ref[...]
    kf = g[:, :c]
    vf = g[:, c:2 * c]
    npos = g[:, 2 * c:]
    q = q_ref[...]
    qrep = jnp.reshape(jnp.broadcast_to(q[:, None, :], (bn, k, c)), (e, c))
    logits = jnp.dot(kf * qrep, ssum_ref[...],
                     preferred_element_type=jnp.float32) * scale
    posrep = jnp.reshape(
        jnp.broadcast_to(pos_ref[...][:, None, :], (bn, k, _POS_PAD)),
        (e, _POS_PAD))
    rel = npos - posrep
    h = _gelu(jnp.dot(rel, w1_ref[...], preferred_element_type=jnp.float32)
              + b1_ref[...])
    z = logits + jnp.dot(h, w2_ref[...], preferred_element_type=jnp.float32) \
        + b2_ref[...]
    z3 = z.reshape(bn, k, _NUM_HEADS)
    zmax = jnp.max(z3, axis=1, keepdims=True)
    ez = jnp.exp(z3 - zmax)
    attn = ez / jnp.sum(ez, axis=1, keepdims=True)
    attn2 = attn.reshape(e, _NUM_HEADS)
    w = jnp.dot(attn2, sbce_ref[...], preferred_element_type=jnp.float32) * vf
    out = jnp.sum(w.reshape(bn, k, c), axis=1)
    y = jnp.dot(out, wo_ref[...], preferred_element_type=jnp.float32) \
        + bo_ref[...] + x_ref[...]
    mean = jnp.mean(y, axis=1, keepdims=True)
    var = jnp.mean((y - mean) ** 2, axis=1, keepdims=True)
    o_ref[...] = (y - mean) * lax.rsqrt(var + 1e-5) * gm_ref[...] + bt_ref[...]


def _sc_gather(tab, idx_flat, n_workers=32, chunk=40):
    e_total = idx_flat.shape[0]
    d = tab.shape[1]
    epw = e_total // n_workers
    assert epw * n_workers == e_total and epw % chunk == 0
    n_chunks = epw // chunk
    mesh = plsc.VectorSubcoreMesh(core_axis_name="c", subcore_axis_name="s")

    @functools.partial(
        pl.kernel,
        out_type=jax.ShapeDtypeStruct((e_total, d), jnp.float32),
        mesh=mesh,
        scratch_types=[
            pltpu.VMEM((epw,), jnp.int32),
            pltpu.VMEM((chunk, d), jnp.float32),
            pltpu.SemaphoreType.DMA,
        ],
        compiler_params=pltpu.CompilerParams(use_tc_tiling_on_sc=False),
    )
    def gather_kernel(tab_hbm, idx_hbm, out_hbm, idx_v, rows_v, sem):
        wid = lax.axis_index("s") * 2 + lax.axis_index("c")
        base = wid * epw
        pltpu.sync_copy(idx_hbm.at[pl.ds(base, epw)], idx_v)

        def body(i, carry):
            off = i * chunk
            pltpu.async_copy(
                tab_hbm.at[idx_v.at[pl.ds(off, chunk)]], rows_v, sem).wait()
            pltpu.sync_copy(rows_v, out_hbm.at[pl.ds(base + off, chunk)])
            return carry

        lax.fori_loop(0, n_chunks, body, 0)

    return gather_kernel(tab, idx_flat)


def kernel(x, pos, idx, Wq, bq, Wk, bk, Wv, bv, Wo, bo, W1, b1, W2, b2,
           gamma, beta):
    n, c = x.shape
    k = idx.shape[1]
    h = _NUM_HEADS
    hd = c // h
    scale = float(hd) ** -0.5
    d_tab = 2 * c + _POS_PAD

    pos_pad = jnp.pad(pos, ((0, 0), (0, _POS_PAD - pos.shape[1])))
    wkv = jnp.concatenate([Wk, Wv], axis=1)
    bkv = jnp.concatenate([bk, bv])[None, :]

    bn_a = 256
    grid_a = (n + bn_a - 1) // bn_a
    q, tab = pl.pallas_call(
        _proj_body,
        grid=(grid_a,),
        in_specs=[
            pl.BlockSpec((bn_a, c), lambda i: (i, 0)),
            pl.BlockSpec((bn_a, _POS_PAD), lambda i: (i, 0)),
            pl.BlockSpec((c, c), lambda i: (0, 0)),
            pl.BlockSpec((1, c), lambda i: (0, 0)),
            pl.BlockSpec((c, 2 * c), lambda i: (0, 0)),
            pl.BlockSpec((1, 2 * c), lambda i: (0, 0)),
        ],
        out_specs=[
            pl.BlockSpec((bn_a, c), lambda i: (i, 0)),
            pl.BlockSpec((bn_a, d_tab), lambda i: (i, 0)),
        ],
        out_shape=[
            jax.ShapeDtypeStruct((n, c), jnp.float32),
            jax.ShapeDtypeStruct((n, d_tab), jnp.float32),
        ],
    )(x, pos_pad, Wq, bq[None, :], wkv, bkv)

    g = _sc_gather(tab, idx.reshape(-1))

    head_ids = jnp.arange(c, dtype=jnp.int32) // hd
    ssum = (head_ids[:, None] == jnp.arange(h, dtype=jnp.int32)[None, :]
            ).astype(jnp.float32)
    sbce = ssum.T
    w1p = jnp.pad(W1, ((0, _POS_PAD - W1.shape[0]), (0, 0)))

    bn_c = 128
    grid_c = (n + bn_c - 1) // bn_c
    y = pl.pallas_call(
        functools.partial(_attn_body, scale, bn_c, k, c),
        grid=(grid_c,),
        in_specs=[
            pl.BlockSpec((bn_c, c), lambda i: (i, 0)),
            pl.BlockSpec((bn_c, c), lambda i: (i, 0)),
            pl.BlockSpec((bn_c, _POS_PAD), lambda i: (i, 0)),
            pl.BlockSpec((bn_c * k, d_tab), lambda i: (i, 0)),
            pl.BlockSpec((c, c), lambda i: (0, 0)),
            pl.BlockSpec((1, c), lambda i: (0, 0)),
            pl.BlockSpec((_POS_PAD, 64), lambda i: (0, 0)),
            pl.BlockSpec((1, 64), lambda i: (0, 0)),
            pl.BlockSpec((64, h), lambda i: (0, 0)),
            pl.BlockSpec((1, h), lambda i: (0, 0)),
            pl.BlockSpec((c, h), lambda i: (0, 0)),
            pl.BlockSpec((h, c), lambda i: (0, 0)),
            pl.BlockSpec((1, c), lambda i: (0, 0)),
            pl.BlockSpec((1, c), lambda i: (0, 0)),
        ],
        out_specs=pl.BlockSpec((bn_c, c), lambda i: (i, 0)),
        out_shape=jax.ShapeDtypeStruct((n, c), jnp.float32),
    )(q, x, pos_pad, g, Wo, bo[None, :], w1p, b1[None, :], W2, b2[None, :],
      ssum, sbce, gamma[None, :], beta[None, :])
    return y


# speedup vs baseline: 3.1164x; 1.8566x over previous
import functools

import jax
import jax.numpy as jnp
from jax import lax
from jax.experimental import pallas as pl
from jax.experimental.pallas import tpu as pltpu
from jax.experimental.pallas import tpu_sc as plsc

_NUM_HEADS = 8
_POS_PAD = 128


def _proj_body(x_ref, pos_ref, wq_ref, bq_ref, wkv_ref, bkv_ref, q_ref, tab_ref):
    xb = x_ref[...]
    q_ref[...] = jnp.dot(xb, wq_ref[...], preferred_element_type=jnp.float32) + bq_ref[...]
    kv = jnp.dot(xb, wkv_ref[...], preferred_element_type=jnp.float32) + bkv_ref[...]
    c = xb.shape[1]
    kb = lax.bitcast_convert_type(kv[:, :c].astype(jnp.bfloat16), jnp.uint16)
    vb = lax.bitcast_convert_type(kv[:, c:].astype(jnp.bfloat16), jnp.uint16)
    kvw = kb.astype(jnp.uint32) | (vb.astype(jnp.uint32) << 16)
    posw = lax.bitcast_convert_type(pos_ref[...], jnp.int32)
    tab_ref[...] = jnp.concatenate(
        [lax.bitcast_convert_type(kvw, jnp.int32), posw], axis=1)


def _erf(x):
    a1, a2, a3, a4, a5 = (0.254829592, -0.284496736, 1.421413741,
                          -1.453152027, 1.061405429)
    p = 0.3275911
    s = jnp.sign(x)
    ax = jnp.abs(x)
    t = 1.0 / (1.0 + p * ax)
    poly = t * (a1 + t * (a2 + t * (a3 + t * (a4 + t * a5))))
    return s * (1.0 - poly * jnp.exp(-ax * ax))


def _gelu(x):
    return 0.5 * x * (1.0 + _erf(x * 0.7071067811865476))


def _attn_body(scale, bn, k, c,
               q_ref, x_ref, pos_ref, g_ref, wo_ref, bo_ref, w1_ref, b1_ref,
               w2_ref, b2_ref, ssum_ref, sbce_ref, gm_ref, bt_ref, o_ref):
    e = bn * k
    g = g_ref[...]
    kvw = g[:, :c]
    kf = lax.bitcast_convert_type(kvw << 16, jnp.float32)
    vf = lax.bitcast_convert_type(
        kvw & jnp.int32(-65536), jnp.float32)
    npos = lax.bitcast_convert_type(g[:, c:], jnp.float32)
    q = q_ref[...]
    qrep = jnp.reshape(jnp.broadcast_to(q[:, None, :], (bn, k, c)), (e, c))
    logits = jnp.dot(kf * qrep, ssum_ref[...],
                     preferred_element_type=jnp.float32) * scale
    posrep = jnp.reshape(
        jnp.broadcast_to(pos_ref[...][:, None, :], (bn, k, _POS_PAD)),
        (e, _POS_PAD))
    rel = npos - posrep
    h = _gelu(jnp.dot(rel, w1_ref[...], preferred_element_type=jnp.float32)
              + b1_ref[...])
    z = logits + jnp.dot(h, w2_ref[...], preferred_element_type=jnp.float32) \
        + b2_ref[...]
    z3 = z.reshape(bn, k, _NUM_HEADS)
    zmax = jnp.max(z3, axis=1, keepdims=True)
    ez = jnp.exp(z3 - zmax)
    attn = ez / jnp.sum(ez, axis=1, keepdims=True)
    attn2 = attn.reshape(e, _NUM_HEADS)
    w = jnp.dot(attn2, sbce_ref[...], preferred_element_type=jnp.float32) * vf
    out = jnp.sum(w.reshape(bn, k, c), axis=1)
    y = jnp.dot(out, wo_ref[...], preferred_element_type=jnp.float32) \
        + bo_ref[...] + x_ref[...]
    mean = jnp.mean(y, axis=1, keepdims=True)
    var = jnp.mean((y - mean) ** 2, axis=1, keepdims=True)
    o_ref[...] = (y - mean) * lax.rsqrt(var + 1e-5) * gm_ref[...] + bt_ref[...]


def _sc_gather(tab, idx_flat, n_workers=32, chunk=40):
    e_total = idx_flat.shape[0]
    d = tab.shape[1]
    epw = e_total // n_workers
    assert epw * n_workers == e_total and epw % chunk == 0
    n_chunks = epw // chunk
    mesh = plsc.VectorSubcoreMesh(core_axis_name="c", subcore_axis_name="s")

    @functools.partial(
        pl.kernel,
        out_type=jax.ShapeDtypeStruct((e_total, d), jnp.int32),
        mesh=mesh,
        scratch_types=[
            pltpu.VMEM((epw,), jnp.int32),
            pltpu.VMEM((chunk, d), jnp.int32),
            pltpu.SemaphoreType.DMA,
        ],
    )
    def gather_kernel(tab_hbm, idx_hbm, out_hbm, idx_v, rows_v, sem):
        wid = lax.axis_index("s") * 2 + lax.axis_index("c")
        base = wid * epw
        pltpu.sync_copy(idx_hbm.at[pl.ds(base, epw)], idx_v)

        def body(i, carry):
            off = i * chunk
            pltpu.async_copy(
                tab_hbm.at[idx_v.at[pl.ds(off, chunk)]], rows_v, sem).wait()
            pltpu.sync_copy(rows_v, out_hbm.at[pl.ds(base + off, chunk)])
            return carry

        lax.fori_loop(0, n_chunks, body, 0)

    return gather_kernel(tab, idx_flat)


def kernel(x, pos, idx, Wq, bq, Wk, bk, Wv, bv, Wo, bo, W1, b1, W2, b2,
           gamma, beta):
    n, c = x.shape
    k = idx.shape[1]
    h = _NUM_HEADS
    hd = c // h
    scale = float(hd) ** -0.5
    d_tab = c + _POS_PAD

    pos_pad = jnp.pad(pos, ((0, 0), (0, _POS_PAD - pos.shape[1])))
    wkv = jnp.concatenate([Wk, Wv], axis=1)
    bkv = jnp.concatenate([bk, bv])[None, :]

    bn_a = 256
    grid_a = (n + bn_a - 1) // bn_a
    q, tab = pl.pallas_call(
        _proj_body,
        grid=(grid_a,),
        in_specs=[
            pl.BlockSpec((bn_a, c), lambda i: (i, 0)),
            pl.BlockSpec((bn_a, _POS_PAD), lambda i: (i, 0)),
            pl.BlockSpec((c, c), lambda i: (0, 0)),
            pl.BlockSpec((1, c), lambda i: (0, 0)),
            pl.BlockSpec((c, 2 * c), lambda i: (0, 0)),
            pl.BlockSpec((1, 2 * c), lambda i: (0, 0)),
        ],
        out_specs=[
            pl.BlockSpec((bn_a, c), lambda i: (i, 0)),
            pl.BlockSpec((bn_a, d_tab), lambda i: (i, 0)),
        ],
        out_shape=[
            jax.ShapeDtypeStruct((n, c), jnp.float32),
            jax.ShapeDtypeStruct((n, d_tab), jnp.int32),
        ],
    )(x, pos_pad, Wq, bq[None, :], wkv, bkv)

    g = _sc_gather(tab, idx.reshape(-1))

    head_ids = jnp.arange(c, dtype=jnp.int32) // hd
    ssum = (head_ids[:, None] == jnp.arange(h, dtype=jnp.int32)[None, :]
            ).astype(jnp.float32)
    sbce = ssum.T
    w1p = jnp.pad(W1, ((0, _POS_PAD - W1.shape[0]), (0, 0)))

    bn_c = 128
    grid_c = (n + bn_c - 1) // bn_c
    y = pl.pallas_call(
        functools.partial(_attn_body, scale, bn_c, k, c),
        grid=(grid_c,),
        in_specs=[
            pl.BlockSpec((bn_c, c), lambda i: (i, 0)),
            pl.BlockSpec((bn_c, c), lambda i: (i, 0)),
            pl.BlockSpec((bn_c, _POS_PAD), lambda i: (i, 0)),
            pl.BlockSpec((bn_c * k, d_tab), lambda i: (i, 0)),
            pl.BlockSpec((c, c), lambda i: (0, 0)),
            pl.BlockSpec((1, c), lambda i: (0, 0)),
            pl.BlockSpec((_POS_PAD, 64), lambda i: (0, 0)),
            pl.BlockSpec((1, 64), lambda i: (0, 0)),
            pl.BlockSpec((64, h), lambda i: (0, 0)),
            pl.BlockSpec((1, h), lambda i: (0, 0)),
            pl.BlockSpec((c, h), lambda i: (0, 0)),
            pl.BlockSpec((h, c), lambda i: (0, 0)),
            pl.BlockSpec((1, c), lambda i: (0, 0)),
            pl.BlockSpec((1, c), lambda i: (0, 0)),
        ],
        out_specs=pl.BlockSpec((bn_c, c), lambda i: (i, 0)),
        out_shape=jax.ShapeDtypeStruct((n, c), jnp.float32),
    )(q, x, pos_pad, g, Wo, bo[None, :], w1p, b1[None, :], W2, b2[None, :],
      ssum, sbce, gamma[None, :], beta[None, :])
    return y


# speedup vs baseline: 4.1345x; 1.3267x over previous
import functools

import jax
import jax.numpy as jnp
from jax import lax
from jax.experimental import pallas as pl
from jax.experimental.pallas import tpu as pltpu
from jax.experimental.pallas import tpu_sc as plsc

_NUM_HEADS = 8
_POS_PAD = 16


def _proj_body(x_ref, wq_ref, bq_ref, wkv_ref, bkv_ref, q_ref, tab_ref):
    xb = x_ref[...]
    q_ref[...] = jnp.dot(xb, wq_ref[...], preferred_element_type=jnp.float32) + bq_ref[...]
    kv = jnp.dot(xb, wkv_ref[...], preferred_element_type=jnp.float32) + bkv_ref[...]
    c = xb.shape[1]
    kb = lax.bitcast_convert_type(kv[:, :c].astype(jnp.bfloat16), jnp.uint16)
    vb = lax.bitcast_convert_type(kv[:, c:].astype(jnp.bfloat16), jnp.uint16)
    kvw = kb.astype(jnp.uint32) | (vb.astype(jnp.uint32) << 16)
    tab_ref[...] = lax.bitcast_convert_type(kvw, jnp.int32)


def _gelu(x):
    u = x * (1.0 + 0.044715 * (x * x))
    return 0.5 * x * (1.0 + jnp.tanh(0.7978845608028654 * u))


def _attn_body(scale, bn, k, c,
               q_ref, x_ref, pos_ref, g_ref, pg_ref, wo_ref, bo_ref, w1_ref,
               b1_ref, w2_ref, b2_ref, ssum_ref, sbce_ref, sbcef_ref, gm_ref,
               bt_ref, o_ref):
    e = bn * k
    kvw = g_ref[...]
    kf = lax.bitcast_convert_type(kvw << 16, jnp.float32)
    vf = lax.bitcast_convert_type(
        kvw & jnp.int32(-65536), jnp.float32)
    q = q_ref[...]
    prod = (kf * q[None, :, :]).reshape(e, c).astype(jnp.bfloat16)
    logits = jnp.dot(prod, ssum_ref[...],
                     preferred_element_type=jnp.float32) * scale
    rel = (pg_ref[...] - pos_ref[...][None, :, :]).reshape(e, _POS_PAD)
    h = _gelu(jnp.dot(rel, w1_ref[...], preferred_element_type=jnp.float32)
              + b1_ref[...])
    z = logits + jnp.dot(h, w2_ref[...], preferred_element_type=jnp.float32) \
        + b2_ref[...]
    z3 = z.reshape(k, bn, _NUM_HEADS)
    zmax = jnp.max(z3, axis=0, keepdims=True)
    ez3 = jnp.exp(z3 - zmax)
    den = jnp.sum(ez3, axis=0)
    ezbc = (jnp.dot(ez3.reshape(e, _NUM_HEADS).astype(jnp.bfloat16),
                    sbce_ref[...], preferred_element_type=jnp.float32)
            .reshape(k, bn, c))
    inv_bc = jnp.dot(1.0 / den, sbcef_ref[...],
                     preferred_element_type=jnp.float32)
    acc = ezbc[0] * vf[0]
    for kk in range(1, k):
        acc = acc + ezbc[kk] * vf[kk]
    out = acc * inv_bc
    y = jnp.dot(out, wo_ref[...], preferred_element_type=jnp.float32) \
        + bo_ref[...] + x_ref[...]
    mean = jnp.mean(y, axis=1, keepdims=True)
    var = jnp.mean((y - mean) ** 2, axis=1, keepdims=True)
    o_ref[...] = (y - mean) * lax.rsqrt(var + 1e-5) * gm_ref[...] + bt_ref[...]


def _sc_gather(tab, idx_flat, e_out, idx_base_fn, dtype, tiled,
               n_workers=32, chunk=40):
    d = tab.shape[1]
    epw = e_out // n_workers
    assert epw * n_workers == e_out and epw % chunk == 0
    n_chunks = epw // chunk
    assert n_chunks >= 4
    mesh = plsc.VectorSubcoreMesh(core_axis_name="c", subcore_axis_name="s")
    params = None if tiled else pltpu.CompilerParams(use_tc_tiling_on_sc=False)

    @functools.partial(
        pl.kernel,
        out_type=jax.ShapeDtypeStruct((e_out, d), dtype),
        mesh=mesh,
        scratch_types=[
            pltpu.VMEM((epw,), jnp.int32),
            pltpu.VMEM((chunk, d), dtype),
            pltpu.VMEM((chunk, d), dtype),
            pltpu.SemaphoreType.DMA,
            pltpu.SemaphoreType.DMA,
            pltpu.SemaphoreType.DMA,
            pltpu.SemaphoreType.DMA,
        ],
        compiler_params=params,
    )
    def gather_kernel(tab_hbm, idx_hbm, out_hbm, idx_v, rows0, rows1,
                      g0, g1, s0, s1):
        wid = lax.axis_index("s") * 2 + lax.axis_index("c")
        base = wid * epw
        pltpu.sync_copy(idx_hbm.at[pl.ds(idx_base_fn(wid), epw)], idx_v)
        rows = (rows0, rows1)
        gsem = (g0, g1)
        ssem = (s0, s1)

        def start_gather(j, b):
            pltpu.async_copy(
                tab_hbm.at[idx_v.at[pl.ds(j * chunk, chunk)]], rows[b],
                gsem[b])

        def wait_gather(j, b):
            pltpu.make_async_copy(
                tab_hbm.at[idx_v.at[pl.ds(j * chunk, chunk)]], rows[b],
                gsem[b]).wait()

        def start_scatter(j, b):
            pltpu.async_copy(
                rows[b], out_hbm.at[pl.ds(base + j * chunk, chunk)], ssem[b])

        def wait_scatter(j, b):
            pltpu.make_async_copy(
                rows[b], out_hbm.at[pl.ds(base + j * chunk, chunk)],
                ssem[b]).wait()

        start_gather(0, 0)

        def pair_body(p, carry):
            j0 = p * 2
            wait_gather(j0, 0)
            start_scatter(j0, 0)

            @pl.when(p > 0)
            def _():
                wait_scatter(j0 - 1, 1)

            start_gather(j0 + 1, 1)
            wait_gather(j0 + 1, 1)
            start_scatter(j0 + 1, 1)
            wait_scatter(j0, 0)

            @pl.when(j0 + 2 < n_chunks)
            def _():
                start_gather(j0 + 2, 0)

            return carry

        lax.fori_loop(0, n_chunks // 2, pair_body, 0)
        if n_chunks % 2 == 1:
            last = n_chunks - 1
            wait_gather(last, 0)
            start_scatter(last, 0)
            wait_scatter(last - 1, 1)
            wait_scatter(last, 0)
        else:
            wait_scatter(n_chunks - 1, 1)

    return gather_kernel(tab, idx_flat)


def kernel(x, pos, idx, Wq, bq, Wk, bk, Wv, bv, Wo, bo, W1, b1, W2, b2,
           gamma, beta):
    n, c = x.shape
    k = idx.shape[1]
    h = _NUM_HEADS
    hd = c // h
    scale = float(hd) ** -0.5

    pos_pad = jnp.pad(pos, ((0, 0), (0, _POS_PAD - pos.shape[1])))
    wkv = jnp.concatenate([Wk, Wv], axis=1)
    bkv = jnp.concatenate([bk, bv])[None, :]
    idx_flat = idx.T.reshape(-1)

    posg = _sc_gather(pos_pad, idx_flat, n * k, lambda w: w * (n * k // 32),
                      jnp.float32, tiled=False)
    pg3 = posg.reshape(k, n, _POS_PAD)

    bn_a = 400
    grid_a = n // bn_a
    q, tab = pl.pallas_call(
        _proj_body,
        grid=(grid_a,),
        in_specs=[
            pl.BlockSpec((bn_a, c), lambda i: (i, 0)),
            pl.BlockSpec((c, c), lambda i: (0, 0)),
            pl.BlockSpec((1, c), lambda i: (0, 0)),
            pl.BlockSpec((c, 2 * c), lambda i: (0, 0)),
            pl.BlockSpec((1, 2 * c), lambda i: (0, 0)),
        ],
        out_specs=[
            pl.BlockSpec((bn_a, c), lambda i: (i, 0)),
            pl.BlockSpec((bn_a, c), lambda i: (i, 0)),
        ],
        out_shape=[
            jax.ShapeDtypeStruct((n, c), jnp.float32),
            jax.ShapeDtypeStruct((n, c), jnp.int32),
        ],
    )(x, Wq, bq[None, :], wkv, bkv)

    head_ids = jnp.arange(c, dtype=jnp.int32) // hd
    ssum = (head_ids[:, None] == jnp.arange(h, dtype=jnp.int32)[None, :]
            ).astype(jnp.float32)
    sbce = ssum.T
    w1p = jnp.pad(W1, ((0, _POS_PAD - W1.shape[0]), (0, 0)))
    ssum_bf = ssum.astype(jnp.bfloat16)
    sbce_bf = sbce.astype(jnp.bfloat16)

    bn_c = 200

    def attn_call(s, t, g3_s):
        n_s = t - s
        grid_c = n_s // bn_c
        off = s // bn_c

        return pl.pallas_call(
            functools.partial(_attn_body, scale, bn_c, k, c),
            grid=(grid_c,),
            in_specs=[
                pl.BlockSpec((bn_c, c), lambda i: (off + i, 0)),
                pl.BlockSpec((bn_c, c), lambda i: (off + i, 0)),
                pl.BlockSpec((bn_c, _POS_PAD), lambda i: (off + i, 0)),
                pl.BlockSpec((k, bn_c, c), lambda i: (0, i, 0)),
                pl.BlockSpec((k, bn_c, _POS_PAD),
                             lambda i: (0, off + i, 0)),
                pl.BlockSpec((c, c), lambda i: (0, 0)),
                pl.BlockSpec((1, c), lambda i: (0, 0)),
                pl.BlockSpec((_POS_PAD, 64), lambda i: (0, 0)),
                pl.BlockSpec((1, 64), lambda i: (0, 0)),
                pl.BlockSpec((64, h), lambda i: (0, 0)),
                pl.BlockSpec((1, h), lambda i: (0, 0)),
                pl.BlockSpec((c, h), lambda i: (0, 0)),
                pl.BlockSpec((h, c), lambda i: (0, 0)),
                pl.BlockSpec((h, c), lambda i: (0, 0)),
                pl.BlockSpec((1, c), lambda i: (0, 0)),
                pl.BlockSpec((1, c), lambda i: (0, 0)),
            ],
            out_specs=pl.BlockSpec((bn_c, c), lambda i: (i, 0)),
            out_shape=jax.ShapeDtypeStruct((n_s, c), jnp.float32),
        )(q, x, pos_pad, g3_s, pg3, Wo, bo[None, :], w1p, b1[None, :], W2,
          b2[None, :], ssum_bf, sbce_bf, sbce, gamma[None, :], beta[None, :])

    bounds = (0, 5200, n)
    ys = []
    for s, t in zip(bounds[:-1], bounds[1:]):
        epw = (t - s) // 2
        g = _sc_gather(
            tab, idx_flat, (t - s) * k,
            lambda w, s=s, epw=epw: (w // 2) * n + s + (w % 2) * epw,
            jnp.int32, tiled=True)
        ys.append(attn_call(s, t, g.reshape(k, t - s, c)))
    return jnp.concatenate(ys, axis=0)


# speedup vs baseline: 4.3595x; 1.0544x over previous
import functools

import jax
import jax.numpy as jnp
from jax import lax
from jax.experimental import pallas as pl
from jax.experimental.pallas import tpu as pltpu
from jax.experimental.pallas import tpu_sc as plsc

_NUM_HEADS = 8
_POS_PAD = 128


def _proj_body(x_ref, pos_ref, wq_ref, bq_ref, wkv_ref, bkv_ref, q_ref,
               tab_ref):
    xb = x_ref[...]
    q_ref[...] = jnp.dot(xb, wq_ref[...], preferred_element_type=jnp.float32) + bq_ref[...]
    kv = jnp.dot(xb, wkv_ref[...], preferred_element_type=jnp.float32) + bkv_ref[...]
    c = xb.shape[1]
    kb = lax.bitcast_convert_type(kv[:, :c].astype(jnp.bfloat16), jnp.uint16)
    vb = lax.bitcast_convert_type(kv[:, c:].astype(jnp.bfloat16), jnp.uint16)
    kvw = kb.astype(jnp.uint32) | (vb.astype(jnp.uint32) << 16)
    posw = lax.bitcast_convert_type(pos_ref[...], jnp.int32)
    tab_ref[...] = jnp.concatenate(
        [lax.bitcast_convert_type(kvw, jnp.int32), posw], axis=1)


def _gelu(x):
    u = x * (1.0 + 0.044715 * (x * x))
    return 0.5 * x * (1.0 + jnp.tanh(0.7978845608028654 * u))


def _attn_body(scale, bn, k, c,
               q_ref, x_ref, pos_ref, g_ref, wo_ref, bo_ref, w1_ref,
               b1_ref, w2_ref, b2_ref, ssum_ref, sbce_ref, sbcef_ref, gm_ref,
               bt_ref, o_ref):
    e = bn * k
    g = g_ref[...]
    kvw = g[:, :, :c]
    kf = lax.bitcast_convert_type(kvw << 16, jnp.float32)
    vf = lax.bitcast_convert_type(
        kvw & jnp.int32(-65536), jnp.float32)
    npos = lax.bitcast_convert_type(g[:, :, c:], jnp.float32)
    q = q_ref[...]
    prod = (kf * q[None, :, :]).reshape(e, c).astype(jnp.bfloat16)
    logits = jnp.dot(prod, ssum_ref[...],
                     preferred_element_type=jnp.float32) * scale
    rel = (npos - pos_ref[...][None, :, :]).reshape(e, _POS_PAD)
    h = _gelu(jnp.dot(rel, w1_ref[...], preferred_element_type=jnp.float32)
              + b1_ref[...])
    z = logits + jnp.dot(h, w2_ref[...], preferred_element_type=jnp.float32) \
        + b2_ref[...]
    z3 = z.reshape(k, bn, _NUM_HEADS)
    zmax = jnp.max(z3, axis=0, keepdims=True)
    ez3 = jnp.exp(z3 - zmax)
    den = jnp.sum(ez3, axis=0)
    ezbc = (jnp.dot(ez3.reshape(e, _NUM_HEADS).astype(jnp.bfloat16),
                    sbce_ref[...], preferred_element_type=jnp.float32)
            .reshape(k, bn, c))
    inv_bc = jnp.dot(1.0 / den, sbcef_ref[...],
                     preferred_element_type=jnp.float32)
    acc = ezbc[0] * vf[0]
    for kk in range(1, k):
        acc = acc + ezbc[kk] * vf[kk]
    out = acc * inv_bc
    y = jnp.dot(out, wo_ref[...], preferred_element_type=jnp.float32) \
        + bo_ref[...] + x_ref[...]
    mean = jnp.mean(y, axis=1, keepdims=True)
    var = jnp.mean((y - mean) ** 2, axis=1, keepdims=True)
    o_ref[...] = (y - mean) * lax.rsqrt(var + 1e-5) * gm_ref[...] + bt_ref[...]


def _sc_gather(tab, idx_flat, e_out, idx_base_fn, dtype, tiled,
               n_workers=32, chunk=40):
    d = tab.shape[1]
    epw = e_out // n_workers
    assert epw * n_workers == e_out and epw % chunk == 0
    n_chunks = epw // chunk
    assert n_chunks >= 4
    mesh = plsc.VectorSubcoreMesh(core_axis_name="c", subcore_axis_name="s")
    params = None if tiled else pltpu.CompilerParams(use_tc_tiling_on_sc=False)

    @functools.partial(
        pl.kernel,
        out_type=jax.ShapeDtypeStruct((e_out, d), dtype),
        mesh=mesh,
        scratch_types=[
            pltpu.VMEM((epw,), jnp.int32),
            pltpu.VMEM((chunk, d), dtype),
            pltpu.VMEM((chunk, d), dtype),
            pltpu.SemaphoreType.DMA,
            pltpu.SemaphoreType.DMA,
            pltpu.SemaphoreType.DMA,
            pltpu.SemaphoreType.DMA,
        ],
        compiler_params=params,
    )
    def gather_kernel(tab_hbm, idx_hbm, out_hbm, idx_v, rows0, rows1,
                      g0, g1, s0, s1):
        wid = lax.axis_index("s") * 2 + lax.axis_index("c")
        base = wid * epw
        pltpu.sync_copy(idx_hbm.at[pl.ds(idx_base_fn(wid), epw)], idx_v)
        rows = (rows0, rows1)
        gsem = (g0, g1)
        ssem = (s0, s1)

        def start_gather(j, b):
            pltpu.async_copy(
                tab_hbm.at[idx_v.at[pl.ds(j * chunk, chunk)]], rows[b],
                gsem[b])

        def wait_gather(j, b):
            pltpu.make_async_copy(
                tab_hbm.at[idx_v.at[pl.ds(j * chunk, chunk)]], rows[b],
                gsem[b]).wait()

        def start_scatter(j, b):
            pltpu.async_copy(
                rows[b], out_hbm.at[pl.ds(base + j * chunk, chunk)], ssem[b])

        def wait_scatter(j, b):
            pltpu.make_async_copy(
                rows[b], out_hbm.at[pl.ds(base + j * chunk, chunk)],
                ssem[b]).wait()

        start_gather(0, 0)

        def pair_body(p, carry):
            j0 = p * 2
            wait_gather(j0, 0)
            start_scatter(j0, 0)

            @pl.when(p > 0)
            def _():
                wait_scatter(j0 - 1, 1)

            start_gather(j0 + 1, 1)
            wait_gather(j0 + 1, 1)
            start_scatter(j0 + 1, 1)
            wait_scatter(j0, 0)

            @pl.when(j0 + 2 < n_chunks)
            def _():
                start_gather(j0 + 2, 0)

            return carry

        lax.fori_loop(0, n_chunks // 2, pair_body, 0)
        if n_chunks % 2 == 1:
            last = n_chunks - 1
            wait_gather(last, 0)
            start_scatter(last, 0)
            wait_scatter(last - 1, 1)
            wait_scatter(last, 0)
        else:
            wait_scatter(n_chunks - 1, 1)

    return gather_kernel(tab, idx_flat)


def kernel(x, pos, idx, Wq, bq, Wk, bk, Wv, bv, Wo, bo, W1, b1, W2, b2,
           gamma, beta):
    n, c = x.shape
    k = idx.shape[1]
    h = _NUM_HEADS
    hd = c // h
    scale = float(hd) ** -0.5

    d_tab = c + _POS_PAD
    pos_pad = jnp.pad(pos, ((0, 0), (0, _POS_PAD - pos.shape[1])))
    wkv = jnp.concatenate([Wk, Wv], axis=1)
    bkv = jnp.concatenate([bk, bv])[None, :]
    idx_flat = idx.T.reshape(-1)

    bn_a = 400
    grid_a = n // bn_a
    q, tab = pl.pallas_call(
        _proj_body,
        grid=(grid_a,),
        in_specs=[
            pl.BlockSpec((bn_a, c), lambda i: (i, 0)),
            pl.BlockSpec((bn_a, _POS_PAD), lambda i: (i, 0)),
            pl.BlockSpec((c, c), lambda i: (0, 0)),
            pl.BlockSpec((1, c), lambda i: (0, 0)),
            pl.BlockSpec((c, 2 * c), lambda i: (0, 0)),
            pl.BlockSpec((1, 2 * c), lambda i: (0, 0)),
        ],
        out_specs=[
            pl.BlockSpec((bn_a, c), lambda i: (i, 0)),
            pl.BlockSpec((bn_a, d_tab), lambda i: (i, 0)),
        ],
        out_shape=[
            jax.ShapeDtypeStruct((n, c), jnp.float32),
            jax.ShapeDtypeStruct((n, d_tab), jnp.int32),
        ],
    )(x, pos_pad, Wq, bq[None, :], wkv, bkv)

    head_ids = jnp.arange(c, dtype=jnp.int32) // hd
    ssum = (head_ids[:, None] == jnp.arange(h, dtype=jnp.int32)[None, :]
            ).astype(jnp.float32)
    sbce = ssum.T
    w1p = jnp.pad(W1, ((0, _POS_PAD - W1.shape[0]), (0, 0)))
    ssum_bf = ssum.astype(jnp.bfloat16)
    sbce_bf = sbce.astype(jnp.bfloat16)

    bn_c = 200

    def attn_call(s, t, g3_s):
        n_s = t - s
        grid_c = n_s // bn_c
        off = s // bn_c

        return pl.pallas_call(
            functools.partial(_attn_body, scale, bn_c, k, c),
            grid=(grid_c,),
            in_specs=[
                pl.BlockSpec((bn_c, c), lambda i: (off + i, 0)),
                pl.BlockSpec((bn_c, c), lambda i: (off + i, 0)),
                pl.BlockSpec((bn_c, _POS_PAD), lambda i: (off + i, 0)),
                pl.BlockSpec((k, bn_c, d_tab), lambda i: (0, i, 0)),
                pl.BlockSpec((c, c), lambda i: (0, 0)),
                pl.BlockSpec((1, c), lambda i: (0, 0)),
                pl.BlockSpec((_POS_PAD, 64), lambda i: (0, 0)),
                pl.BlockSpec((1, 64), lambda i: (0, 0)),
                pl.BlockSpec((64, h), lambda i: (0, 0)),
                pl.BlockSpec((1, h), lambda i: (0, 0)),
                pl.BlockSpec((c, h), lambda i: (0, 0)),
                pl.BlockSpec((h, c), lambda i: (0, 0)),
                pl.BlockSpec((h, c), lambda i: (0, 0)),
                pl.BlockSpec((1, c), lambda i: (0, 0)),
                pl.BlockSpec((1, c), lambda i: (0, 0)),
            ],
            out_specs=pl.BlockSpec((bn_c, c), lambda i: (i, 0)),
            out_shape=jax.ShapeDtypeStruct((n_s, c), jnp.float32),
        )(q, x, pos_pad, g3_s, Wo, bo[None, :], w1p, b1[None, :], W2,
          b2[None, :], ssum_bf, sbce_bf, sbce, gamma[None, :], beta[None, :])

    bounds = (0, 5200, n)
    ys = []
    for s, t in zip(bounds[:-1], bounds[1:]):
        epw = (t - s) // 2
        g = _sc_gather(
            tab, idx_flat, (t - s) * k,
            lambda w, s=s, epw=epw: (w // 2) * n + s + (w % 2) * epw,
            jnp.int32, tiled=True)
        ys.append(attn_call(s, t, g.reshape(k, t - s, d_tab)))
    return jnp.concatenate(ys, axis=0)


# speedup vs baseline: 4.6296x; 1.0620x over previous
import functools

import jax
import jax.numpy as jnp
from jax import lax
from jax.experimental import pallas as pl
from jax.experimental.pallas import tpu as pltpu
from jax.experimental.pallas import tpu_sc as plsc

_NUM_HEADS = 8
_POS_PAD = 128


def _proj_body(x_ref, pos_ref, wq_ref, bq_ref, wkv_ref, bkv_ref, q_ref,
               tab_ref):
    xb = x_ref[...]
    q_ref[...] = jnp.dot(xb, wq_ref[...], preferred_element_type=jnp.float32) + bq_ref[...]
    kv = jnp.dot(xb, wkv_ref[...], preferred_element_type=jnp.float32) + bkv_ref[...]
    c = xb.shape[1]
    kb = lax.bitcast_convert_type(kv[:, :c].astype(jnp.bfloat16), jnp.uint16)
    vb = lax.bitcast_convert_type(kv[:, c:].astype(jnp.bfloat16), jnp.uint16)
    kvw = kb.astype(jnp.uint32) | (vb.astype(jnp.uint32) << 16)
    posw = lax.bitcast_convert_type(pos_ref[...], jnp.int32)
    tab_ref[...] = jnp.concatenate(
        [lax.bitcast_convert_type(kvw, jnp.int32), posw], axis=1)


def _gelu(x):
    u = x * (1.0 + 0.044715 * (x * x))
    return 0.5 * x * (1.0 + jnp.tanh(0.7978845608028654 * u))


def _attn_body(scale, bn, k, c,
               q_ref, x_ref, pos_ref, g_ref, wo_ref, bo_ref, w1_ref,
               b1_ref, w2_ref, b2_ref, ssum_ref, sbce_ref, sbcef_ref, gm_ref,
               bt_ref, o_ref):
    e = bn * k
    g = g_ref[...]
    kvw = g[:, :, :c]
    kf = lax.bitcast_convert_type(kvw << 16, jnp.float32)
    vf = lax.bitcast_convert_type(
        kvw & jnp.int32(-65536), jnp.float32)
    npos = lax.bitcast_convert_type(g[:, :, c:], jnp.float32)
    q = q_ref[...]
    prod = (kf * q[None, :, :]).reshape(e, c).astype(jnp.bfloat16)
    logits = jnp.dot(prod, ssum_ref[...],
                     preferred_element_type=jnp.float32) * scale
    rel = (npos - pos_ref[...][None, :, :]).reshape(e, _POS_PAD)
    h = _gelu(jnp.dot(rel, w1_ref[...], preferred_element_type=jnp.float32)
              + b1_ref[...])
    z = logits + jnp.dot(h, w2_ref[...], preferred_element_type=jnp.float32) \
        + b2_ref[...]
    z3 = z.reshape(k, bn, _NUM_HEADS)
    zmax = jnp.max(z3, axis=0, keepdims=True)
    ez3 = jnp.exp(z3 - zmax)
    den = jnp.sum(ez3, axis=0)
    ezbc = (jnp.dot(ez3.reshape(e, _NUM_HEADS).astype(jnp.bfloat16),
                    sbce_ref[...], preferred_element_type=jnp.float32)
            .reshape(k, bn, c))
    inv_bc = jnp.dot(1.0 / den, sbcef_ref[...],
                     preferred_element_type=jnp.float32)
    acc = ezbc[0] * vf[0]
    for kk in range(1, k):
        acc = acc + ezbc[kk] * vf[kk]
    out = acc * inv_bc
    y = jnp.dot(out, wo_ref[...], preferred_element_type=jnp.float32) \
        + bo_ref[...] + x_ref[...]
    mean = jnp.mean(y, axis=1, keepdims=True)
    var = jnp.mean((y - mean) ** 2, axis=1, keepdims=True)
    o_ref[...] = (y - mean) * lax.rsqrt(var + 1e-5) * gm_ref[...] + bt_ref[...]


def _pick_chunk(epw):
    for c in range(128, 0, -8):
        if epw % c == 0:
            return c
    raise ValueError(epw)


def _sc_gather(tab, idx_flat, e_out, idx_base_fn, dtype, tiled,
               n_workers=32, chunk=None):
    d = tab.shape[1]
    epw = e_out // n_workers
    if chunk is None:
        chunk = _pick_chunk(epw)
    assert epw * n_workers == e_out and epw % chunk == 0
    n_chunks = epw // chunk
    assert n_chunks >= 4
    mesh = plsc.VectorSubcoreMesh(core_axis_name="c", subcore_axis_name="s")
    params = None if tiled else pltpu.CompilerParams(use_tc_tiling_on_sc=False)

    @functools.partial(
        pl.kernel,
        out_type=jax.ShapeDtypeStruct((e_out, d), dtype),
        mesh=mesh,
        scratch_types=[
            pltpu.VMEM((epw,), jnp.int32),
            pltpu.VMEM((chunk, d), dtype),
            pltpu.VMEM((chunk, d), dtype),
            pltpu.SemaphoreType.DMA,
            pltpu.SemaphoreType.DMA,
            pltpu.SemaphoreType.DMA,
            pltpu.SemaphoreType.DMA,
        ],
        compiler_params=params,
    )
    def gather_kernel(tab_hbm, idx_hbm, out_hbm, idx_v, rows0, rows1,
                      g0, g1, s0, s1):
        wid = lax.axis_index("s") * 2 + lax.axis_index("c")
        base = wid * epw
        pltpu.sync_copy(idx_hbm.at[pl.ds(idx_base_fn(wid), epw)], idx_v)
        rows = (rows0, rows1)
        gsem = (g0, g1)
        ssem = (s0, s1)

        def start_gather(j, b):
            pltpu.async_copy(
                tab_hbm.at[idx_v.at[pl.ds(j * chunk, chunk)]], rows[b],
                gsem[b])

        def wait_gather(j, b):
            pltpu.make_async_copy(
                tab_hbm.at[idx_v.at[pl.ds(j * chunk, chunk)]], rows[b],
                gsem[b]).wait()

        def start_scatter(j, b):
            pltpu.async_copy(
                rows[b], out_hbm.at[pl.ds(base + j * chunk, chunk)], ssem[b])

        def wait_scatter(j, b):
            pltpu.make_async_copy(
                rows[b], out_hbm.at[pl.ds(base + j * chunk, chunk)],
                ssem[b]).wait()

        start_gather(0, 0)

        def pair_body(p, carry):
            j0 = p * 2
            wait_gather(j0, 0)
            start_scatter(j0, 0)

            @pl.when(p > 0)
            def _():
                wait_scatter(j0 - 1, 1)

            start_gather(j0 + 1, 1)
            wait_gather(j0 + 1, 1)
            start_scatter(j0 + 1, 1)
            wait_scatter(j0, 0)

            @pl.when(j0 + 2 < n_chunks)
            def _():
                start_gather(j0 + 2, 0)

            return carry

        lax.fori_loop(0, n_chunks // 2, pair_body, 0)
        if n_chunks % 2 == 1:
            last = n_chunks - 1
            wait_gather(last, 0)
            start_scatter(last, 0)
            wait_scatter(last - 1, 1)
            wait_scatter(last, 0)
        else:
            wait_scatter(n_chunks - 1, 1)

    return gather_kernel(tab, idx_flat)


def kernel(x, pos, idx, Wq, bq, Wk, bk, Wv, bv, Wo, bo, W1, b1, W2, b2,
           gamma, beta):
    n, c = x.shape
    k = idx.shape[1]
    h = _NUM_HEADS
    hd = c // h
    scale = float(hd) ** -0.5

    d_tab = c + _POS_PAD
    pos_pad = jnp.pad(pos, ((0, 0), (0, _POS_PAD - pos.shape[1])))
    wkv = jnp.concatenate([Wk, Wv], axis=1)
    bkv = jnp.concatenate([bk, bv])[None, :]
    idx_flat = idx.T.reshape(-1)

    bn_a = 400
    grid_a = n // bn_a
    q, tab = pl.pallas_call(
        _proj_body,
        grid=(grid_a,),
        in_specs=[
            pl.BlockSpec((bn_a, c), lambda i: (i, 0)),
            pl.BlockSpec((bn_a, _POS_PAD), lambda i: (i, 0)),
            pl.BlockSpec((c, c), lambda i: (0, 0)),
            pl.BlockSpec((1, c), lambda i: (0, 0)),
            pl.BlockSpec((c, 2 * c), lambda i: (0, 0)),
            pl.BlockSpec((1, 2 * c), lambda i: (0, 0)),
        ],
        out_specs=[
            pl.BlockSpec((bn_a, c), lambda i: (i, 0)),
            pl.BlockSpec((bn_a, d_tab), lambda i: (i, 0)),
        ],
        out_shape=[
            jax.ShapeDtypeStruct((n, c), jnp.float32),
            jax.ShapeDtypeStruct((n, d_tab), jnp.int32),
        ],
    )(x, pos_pad, Wq, bq[None, :], wkv, bkv)

    head_ids = jnp.arange(c, dtype=jnp.int32) // hd
    ssum = (head_ids[:, None] == jnp.arange(h, dtype=jnp.int32)[None, :]
            ).astype(jnp.float32)
    sbce = ssum.T
    w1p = jnp.pad(W1, ((0, _POS_PAD - W1.shape[0]), (0, 0)))
    ssum_bf = ssum.astype(jnp.bfloat16)
    sbce_bf = sbce.astype(jnp.bfloat16)

    bn_c = 200

    def attn_call(s, t, g3_s):
        n_s = t - s
        grid_c = n_s // bn_c
        off = s // bn_c

        return pl.pallas_call(
            functools.partial(_attn_body, scale, bn_c, k, c),
            grid=(grid_c,),
            in_specs=[
                pl.BlockSpec((bn_c, c), lambda i: (off + i, 0)),
                pl.BlockSpec((bn_c, c), lambda i: (off + i, 0)),
                pl.BlockSpec((bn_c, _POS_PAD), lambda i: (off + i, 0)),
                pl.BlockSpec((k, bn_c, d_tab), lambda i: (0, i, 0)),
                pl.BlockSpec((c, c), lambda i: (0, 0)),
                pl.BlockSpec((1, c), lambda i: (0, 0)),
                pl.BlockSpec((_POS_PAD, 64), lambda i: (0, 0)),
                pl.BlockSpec((1, 64), lambda i: (0, 0)),
                pl.BlockSpec((64, h), lambda i: (0, 0)),
                pl.BlockSpec((1, h), lambda i: (0, 0)),
                pl.BlockSpec((c, h), lambda i: (0, 0)),
                pl.BlockSpec((h, c), lambda i: (0, 0)),
                pl.BlockSpec((h, c), lambda i: (0, 0)),
                pl.BlockSpec((1, c), lambda i: (0, 0)),
                pl.BlockSpec((1, c), lambda i: (0, 0)),
            ],
            out_specs=pl.BlockSpec((bn_c, c), lambda i: (i, 0)),
            out_shape=jax.ShapeDtypeStruct((n_s, c), jnp.float32),
        )(q, x, pos_pad, g3_s, Wo, bo[None, :], w1p, b1[None, :], W2,
          b2[None, :], ssum_bf, sbce_bf, sbce, gamma[None, :], beta[None, :])

    bounds = (0, 6000, n)
    ys = []
    for s, t in zip(bounds[:-1], bounds[1:]):
        epw = (t - s) // 2
        g = _sc_gather(
            tab, idx_flat, (t - s) * k,
            lambda w, s=s, epw=epw: (w // 2) * n + s + (w % 2) * epw,
            jnp.int32, tiled=True)
        ys.append(attn_call(s, t, g.reshape(k, t - s, d_tab)))
    return jnp.concatenate(ys, axis=0)
